# baseline (speedup 1.0000x reference)
import functools

import jax
import jax.numpy as jnp
from jax import lax
from jax.experimental import pallas as pl
from jax.experimental.pallas import tpu as pltpu
from jax.experimental.pallas import tpu_sc as plsc

DIM_VSA = 2048
DIM_NODE = 27
DIM_NODE_PAD = 32
DIM_EDGE = 12
MAX_EDGES = 128
B = 64
N = 128
TWO_PI = 2.0 * jnp.pi
HALF_B = 32


def _remap_phase(x):
    return x - TWO_PI * jnp.round(x / TWO_PI)


def _sc_body(half, mask_hbm, atoms_hbm, pairs_hbm,
             src_out, dst_out, edge_out, valid_out,
             mask_v, idx_v,
             sidx_v, didx_v, sloc_v, dloc_v, val_v, srow_v, drow_v, erow_v,
             pci0, pci1, pci2, pci3, pb0, pb1, pb2, pb3,
             asem, ps0, ps1, ps2, ps3):
    wid = lax.axis_index("s") * 2 + lax.axis_index("c")

    if True:
        b = half * HALF_B + wid
        pltpu.sync_copy(mask_hbm.at[b], mask_v)

        fill = jnp.full((16,), N * N - 1, jnp.int32)
        for t in range(10):
            idx_v[pl.ds(t * 16, 16)] = fill

        def row_body(i, cntv):
            iv = jnp.broadcast_to(i, (16,))
            keeps, flats, pcss = [], [], []
            for c in range(8):
                m = mask_v[i, pl.ds(c * 16, 16)]
                j16 = lax.iota(jnp.int32, 16) + c * 16
                keep = (m != 0.0) & (j16 > iv)
                keeps.append(keep)
                flats.append(iv * N + j16)
                pcss.append(plsc.cumsum(keep.astype(jnp.int32)))
            base = cntv
            for c in range(8):
                pos = jnp.where(keeps[c],
                                jnp.minimum(base + pcss[c] - 1, 159), 159)
                plsc.store_scatter(idx_v, [pos], flats[c])
                base = base + jnp.broadcast_to(pcss[c][15], (16,))
            return base

        cntv0 = jnp.zeros((16,), jnp.int32)
        cntv = lax.fori_loop(0, N, row_body, cntv0)
        cnt = jnp.minimum(cntv[0], MAX_EDGES)

        zero16 = jnp.zeros((16,), jnp.float32)
        for t in range(8):
            fidx = idx_v[pl.ds(t * 16, 16)]
            s = lax.shift_right_logical(fidx, 7)
            d = lax.bitwise_and(fidx, N - 1)
            lane = lax.iota(jnp.int32, 16) + t * 16
            cntv = jnp.broadcast_to(cnt, (16,))
            ok = lane < cntv
            sidx_v[pl.ds(t * 16, 16)] = jnp.where(ok, b * N + s, B * N)
            didx_v[pl.ds(t * 16, 16)] = jnp.where(ok, b * N + d, B * N)
            sloc_v[pl.ds(t * 16, 16)] = s
            dloc_v[pl.ds(t * 16, 16)] = d
            val_v[pl.ds(t * 16, 16)] = jnp.where(ok, 1.0, 0.0)
            for c in range(DIM_EDGE, 16):
                plsc.store_scatter(
                    erow_v, [lane, jnp.broadcast_to(jnp.int32(c), (16,))],
                    zero16)

        cp1 = pltpu.async_copy(atoms_hbm.at[sidx_v], srow_v, asem)
        cp2 = pltpu.async_copy(atoms_hbm.at[didx_v], drow_v, asem)

    pcis = [pci0, pci1, pci2, pci3]
    pbs = [pb0, pb1, pb2, pb3]
    pss = [ps0, ps1, ps2, ps3]
    inflight = [None] * 4

    def _issue(c):
        slot = c % 4
        cbase = (b * DIM_EDGE + c) * N
        for t in range(8):
            s16 = sloc_v[pl.ds(t * 16, 16)]
            pcis[slot][pl.ds(t * 16, 16)] = cbase + s16
        inflight[slot] = pltpu.async_copy(pairs_hbm.at[pcis[slot]], pbs[slot],
                                          pss[slot])

    for c in range(4):
        _issue(c)
    for c in range(DIM_EDGE):
        inflight[c % 4].wait()
        buf = pbs[c % 4]
        cc = jnp.broadcast_to(jnp.int32(c), (16,))
        for t in range(8):
            e16 = lax.iota(jnp.int32, 16) + t * 16
            d16 = dloc_v[pl.ds(t * 16, 16)]
            v = plsc.load_gather(buf, [e16, d16])
            okv = val_v[pl.ds(t * 16, 16)]
            plsc.store_scatter(erow_v, [e16, cc], v * okv)
        if c + 4 < DIM_EDGE:
            _issue(c + 4)

    cp1.wait()
    cp2.wait()
    pltpu.sync_copy(srow_v, src_out.at[wid])
    pltpu.sync_copy(drow_v, dst_out.at[wid])
    pltpu.sync_copy(erow_v, edge_out.at[wid])
    pltpu.sync_copy(val_v, valid_out.at[wid])


def _sc_extract(pair_mask, atoms_flat, pairs_flat, half):
    mesh = plsc.VectorSubcoreMesh(core_axis_name="c", subcore_axis_name="s")
    f32 = jnp.float32
    run = pl.kernel(
        functools.partial(_sc_body, half),
        out_type=(
            jax.ShapeDtypeStruct((HALF_B, MAX_EDGES, DIM_NODE_PAD), f32),
            jax.ShapeDtypeStruct((HALF_B, MAX_EDGES, DIM_NODE_PAD), f32),
            jax.ShapeDtypeStruct((HALF_B, MAX_EDGES, 16), f32),
            jax.ShapeDtypeStruct((HALF_B, MAX_EDGES), f32),
        ),
        mesh=mesh,
        scratch_types=(
            (pltpu.VMEM((N, N), f32),)
            + (pltpu.VMEM((160,), jnp.int32),)
            + (pltpu.VMEM((MAX_EDGES,), jnp.int32),
               pltpu.VMEM((MAX_EDGES,), jnp.int32),
               pltpu.VMEM((MAX_EDGES,), jnp.int32),
               pltpu.VMEM((MAX_EDGES,), jnp.int32),
               pltpu.VMEM((MAX_EDGES,), f32),
               pltpu.VMEM((MAX_EDGES, DIM_NODE_PAD), f32),
               pltpu.VMEM((MAX_EDGES, DIM_NODE_PAD), f32),
               pltpu.VMEM((MAX_EDGES, 16), f32))
            + 4 * (pltpu.VMEM((MAX_EDGES,), jnp.int32),)
            + 4 * (pltpu.VMEM((MAX_EDGES, N), f32),)
            + 5 * (pltpu.SemaphoreType.DMA,)
        ),
        compiler_params=pltpu.CompilerParams(
            needs_layout_passes=False, use_tc_tiling_on_sc=False),
    )
    return run(pair_mask, atoms_flat, pairs_flat)


TC_ROWS = 256


def _tc_body(s_ref, d_ref, e_ref, v_ref, pa_ref, pb_ref, f_ref, out_ref,
             spec_ref):
    @pl.when(pl.program_id(0) == 0)
    def _():
        pos = lax.broadcasted_iota(jnp.int32, (TC_ROWS, DIM_VSA), 0)
        pos = lax.rem(pos, MAX_EDGES).astype(jnp.float32)
        spec_ref[...] = _remap_phase(pos * f_ref[...])

    acc = jnp.dot(s_ref[...], pa_ref[...], preferred_element_type=jnp.float32)
    acc += jnp.dot(d_ref[...], pa_ref[...], preferred_element_type=jnp.float32)
    acc += jnp.dot(e_ref[...], pb_ref[...], preferred_element_type=jnp.float32)
    g = _remap_phase(acc)
    g = _remap_phase(g + spec_ref[...])
    out_ref[...] = g * v_ref[...]


def _tc_compute_half(src_rows, dst_rows, edge_rows, valid2, pa_pad, pb_pad,
                     fb, half, prev):
    rows = B * MAX_EDGES
    half_grid = (HALF_B * MAX_EDGES) // TC_ROWS
    off = half * half_grid

    in_specs = [
        pl.BlockSpec((TC_ROWS, DIM_NODE_PAD), lambda g: (g, 0)),
        pl.BlockSpec((TC_ROWS, DIM_NODE_PAD), lambda g: (g, 0)),
        pl.BlockSpec((TC_ROWS, 16), lambda g: (g, 0)),
        pl.BlockSpec((TC_ROWS, 1), lambda g: (g, 0)),
        pl.BlockSpec((DIM_NODE_PAD, DIM_VSA), lambda g: (0, 0)),
        pl.BlockSpec((16, DIM_VSA), lambda g: (0, 0)),
        pl.BlockSpec((1, DIM_VSA), lambda g: (0, 0)),
    ]
    args = [src_rows, dst_rows, edge_rows, valid2, pa_pad, pb_pad, fb]
    kwargs = {}
    if prev is not None:
        in_specs.append(pl.BlockSpec(memory_space=pl.ANY))
        args.append(prev)
        kwargs["input_output_aliases"] = {7: 0}

    def body(*refs):
        if prev is not None:
            s, d, e, v, pa, pb, f, _prev, out, spec = refs
        else:
            s, d, e, v, pa, pb, f, out, spec = refs
        _tc_body(s, d, e, v, pa, pb, f, out, spec)

    return pl.pallas_call(
        body,
        grid=(half_grid,),
        in_specs=in_specs,
        out_specs=pl.BlockSpec((TC_ROWS, DIM_VSA), lambda g: (g + off, 0)),
        out_shape=jax.ShapeDtypeStruct((rows, DIM_VSA), jnp.float32),
        scratch_shapes=[pltpu.VMEM((TC_ROWS, DIM_VSA), jnp.float32)],
        compiler_params=pltpu.CompilerParams(
            dimension_semantics=("arbitrary",),
        ),
        **kwargs,
    )(*args)


def kernel(atoms, pairs, pair_mask, active, atom_projection, bond_projection,
           frequency_basis):
    atoms_pad = jnp.pad(atoms, ((0, 0), (0, 0), (0, DIM_NODE_PAD - DIM_NODE)))
    atoms_flat = jnp.pad(atoms_pad.reshape(B * N, DIM_NODE_PAD),
                         ((0, 8), (0, 0)))
    pairs_rows = jnp.transpose(pairs, (0, 3, 1, 2)).reshape(
        B * DIM_EDGE * N, N)
    pa_pad = jnp.pad(atom_projection, ((0, DIM_NODE_PAD - DIM_NODE), (0, 0)))
    pb_pad = jnp.pad(bond_projection, ((0, 16 - DIM_EDGE), (0, 0)))

    hrows = HALF_B * MAX_EDGES
    out = None
    for half in range(2):
        s_h, d_h, e_h, v_h = _sc_extract(pair_mask, atoms_flat, pairs_rows,
                                         half)
        out = _tc_compute_half(s_h.reshape(hrows, DIM_NODE_PAD),
                               d_h.reshape(hrows, DIM_NODE_PAD),
                               e_h.reshape(hrows, 16),
                               v_h.reshape(hrows, 1), pa_pad, pb_pad,
                               frequency_basis, half, out)
    return out.reshape(B, MAX_EDGES, DIM_VSA)


# speedup vs baseline: 1.0426x; 1.0426x over previous
import functools

import jax
import jax.numpy as jnp
from jax import lax
from jax.experimental import pallas as pl
from jax.experimental.pallas import tpu as pltpu
from jax.experimental.pallas import tpu_sc as plsc

DIM_VSA = 2048
DIM_NODE = 27
DIM_NODE_PAD = 32
DIM_EDGE = 12
MAX_EDGES = 128
B = 64
N = 128
TWO_PI = 2.0 * jnp.pi
GRAPHS_PER_WORKER = 2


def _remap_phase(x):
    return x - TWO_PI * jnp.round(x / TWO_PI)


def _sc_body(mask_hbm, atoms_hbm, pairs_hbm,
             src_out, dst_out, edge_out, valid_out,
             mask_v, idx_v,
             sidx0, didx0, sloc0, dloc0, val0, srow0, drow0, erow0,
             sidx1, didx1, sloc1, dloc1, val1, srow1, drow1, erow1,
             pci0, pci1, pci2, pci3, pb0, pb1, pb2, pb3,
             asem, ps0, ps1, ps2, ps3):
    wid = lax.axis_index("s") * 2 + lax.axis_index("c")
    SIDX = [sidx0, sidx1]
    DIDX = [didx0, didx1]
    SLOC = [sloc0, sloc1]
    DLOC = [dloc0, dloc1]
    VAL = [val0, val1]
    SROW = [srow0, srow1]
    DROW = [drow0, drow1]
    EROW = [erow0, erow1]
    acps = []

    for k in range(GRAPHS_PER_WORKER):
        b = wid * GRAPHS_PER_WORKER + k
        sidx_v, didx_v = SIDX[k], DIDX[k]
        sloc_v, dloc_v, val_v, erow_v = SLOC[k], DLOC[k], VAL[k], EROW[k]
        pltpu.sync_copy(mask_hbm.at[b], mask_v)

        fill = jnp.full((16,), N * N - 1, jnp.int32)
        for t in range(10):
            idx_v[pl.ds(t * 16, 16)] = fill

        def row_body(i, cntv):
            iv = jnp.broadcast_to(i, (16,))
            keeps, flats, pcss = [], [], []
            for c in range(8):
                m = mask_v[i, pl.ds(c * 16, 16)]
                j16 = lax.iota(jnp.int32, 16) + c * 16
                keep = (m != 0.0) & (j16 > iv)
                keeps.append(keep)
                flats.append(iv * N + j16)
                pcss.append(plsc.cumsum(keep.astype(jnp.int32)))
            base = cntv
            for c in range(8):
                pos = jnp.where(keeps[c],
                                jnp.minimum(base + pcss[c] - 1, 159), 159)
                plsc.store_scatter(idx_v, [pos], flats[c])
                base = base + jnp.broadcast_to(pcss[c][15], (16,))
            return base

        cntv0 = jnp.zeros((16,), jnp.int32)
        cntv = lax.fori_loop(0, N, row_body, cntv0)
        cnt = jnp.minimum(cntv[0], MAX_EDGES)

        zero16 = jnp.zeros((16,), jnp.float32)
        for t in range(8):
            fidx = idx_v[pl.ds(t * 16, 16)]
            s = lax.shift_right_logical(fidx, 7)
            d = lax.bitwise_and(fidx, N - 1)
            lane = lax.iota(jnp.int32, 16) + t * 16
            cntv = jnp.broadcast_to(cnt, (16,))
            ok = lane < cntv
            sidx_v[pl.ds(t * 16, 16)] = jnp.where(ok, b * N + s, B * N)
            didx_v[pl.ds(t * 16, 16)] = jnp.where(ok, b * N + d, B * N)
            sloc_v[pl.ds(t * 16, 16)] = s
            dloc_v[pl.ds(t * 16, 16)] = d
            val_v[pl.ds(t * 16, 16)] = jnp.where(ok, 1.0, 0.0)
            for c in range(DIM_EDGE, 16):
                plsc.store_scatter(
                    erow_v, [lane, jnp.broadcast_to(jnp.int32(c), (16,))],
                    zero16)

        acps.append(pltpu.async_copy(atoms_hbm.at[sidx_v], SROW[k], asem))
        acps.append(pltpu.async_copy(atoms_hbm.at[didx_v], DROW[k], asem))

    pcis = [pci0, pci1, pci2, pci3]
    pbs = [pb0, pb1, pb2, pb3]
    pss = [ps0, ps1, ps2, ps3]
    inflight = [None] * 4
    tasks = [(k, c) for c in range(DIM_EDGE) for k in range(GRAPHS_PER_WORKER)]

    def _issue(i):
        k, c = tasks[i]
        slot = i % 4
        base = ((wid * GRAPHS_PER_WORKER + k) * DIM_EDGE + c) * N
        for t in range(8):
            s16 = SLOC[k][pl.ds(t * 16, 16)]
            pcis[slot][pl.ds(t * 16, 16)] = base + s16
        inflight[slot] = pltpu.async_copy(pairs_hbm.at[pcis[slot]], pbs[slot],
                                          pss[slot])

    for i in range(4):
        _issue(i)
    for i in range(len(tasks)):
        k, c = tasks[i]
        inflight[i % 4].wait()
        buf = pbs[i % 4]
        cc = jnp.broadcast_to(jnp.int32(c), (16,))
        for t in range(8):
            e16 = lax.iota(jnp.int32, 16) + t * 16
            d16 = DLOC[k][pl.ds(t * 16, 16)]
            v = plsc.load_gather(buf, [e16, d16])
            okv = VAL[k][pl.ds(t * 16, 16)]
            plsc.store_scatter(EROW[k], [e16, cc], v * okv)
        if i + 4 < len(tasks):
            _issue(i + 4)

    for cp in acps:
        cp.wait()
    for k in range(GRAPHS_PER_WORKER):
        b = wid * GRAPHS_PER_WORKER + k
        pltpu.sync_copy(SROW[k], src_out.at[b])
        pltpu.sync_copy(DROW[k], dst_out.at[b])
        pltpu.sync_copy(EROW[k], edge_out.at[b])
        pltpu.sync_copy(VAL[k], valid_out.at[b])


def _sc_extract(pair_mask, atoms_flat, pairs_flat):
    mesh = plsc.VectorSubcoreMesh(core_axis_name="c", subcore_axis_name="s")
    f32 = jnp.float32
    run = pl.kernel(
        _sc_body,
        out_type=(
            jax.ShapeDtypeStruct((B, MAX_EDGES, DIM_NODE_PAD), f32),
            jax.ShapeDtypeStruct((B, MAX_EDGES, DIM_NODE_PAD), f32),
            jax.ShapeDtypeStruct((B, MAX_EDGES, 16), f32),
            jax.ShapeDtypeStruct((B, MAX_EDGES), f32),
        ),
        mesh=mesh,
        scratch_types=(
            (pltpu.VMEM((N, N), f32),)
            + (pltpu.VMEM((160,), jnp.int32),)
            + 2 * (pltpu.VMEM((MAX_EDGES,), jnp.int32),
                   pltpu.VMEM((MAX_EDGES,), jnp.int32),
                   pltpu.VMEM((MAX_EDGES,), jnp.int32),
                   pltpu.VMEM((MAX_EDGES,), jnp.int32),
                   pltpu.VMEM((MAX_EDGES,), f32),
                   pltpu.VMEM((MAX_EDGES, DIM_NODE_PAD), f32),
                   pltpu.VMEM((MAX_EDGES, DIM_NODE_PAD), f32),
                   pltpu.VMEM((MAX_EDGES, 16), f32))
            + 4 * (pltpu.VMEM((MAX_EDGES,), jnp.int32),)
            + 4 * (pltpu.VMEM((MAX_EDGES, N), f32),)
            + 5 * (pltpu.SemaphoreType.DMA,)
        ),
        compiler_params=pltpu.CompilerParams(
            needs_layout_passes=False, use_tc_tiling_on_sc=False),
    )
    return run(pair_mask, atoms_flat, pairs_flat)


TC_ROWS = 512


def _tc_body(s_ref, d_ref, e_ref, v_ref, pa_ref, pb_ref, f_ref, out_ref,
             spec_ref):
    @pl.when(pl.program_id(0) == 0)
    def _():
        pos = lax.broadcasted_iota(jnp.int32, (TC_ROWS, DIM_VSA), 0)
        pos = lax.rem(pos, MAX_EDGES).astype(jnp.float32)
        spec_ref[...] = _remap_phase(pos * f_ref[...])

    acc = jnp.dot(s_ref[...], pa_ref[...], preferred_element_type=jnp.float32)
    acc += jnp.dot(d_ref[...], pa_ref[...], preferred_element_type=jnp.float32)
    acc += jnp.dot(e_ref[...], pb_ref[...], preferred_element_type=jnp.float32)
    g = _remap_phase(acc)
    g = _remap_phase(g + spec_ref[...])
    out_ref[...] = g * v_ref[...]


def _tc_compute(src_rows, dst_rows, edge_rows, valid2, pa_pad, pb_pad, fb):
    rows = B * MAX_EDGES
    grid = (rows // TC_ROWS,)
    return pl.pallas_call(
        _tc_body,
        grid=grid,
        in_specs=[
            pl.BlockSpec((TC_ROWS, DIM_NODE_PAD), lambda g: (g, 0)),
            pl.BlockSpec((TC_ROWS, DIM_NODE_PAD), lambda g: (g, 0)),
            pl.BlockSpec((TC_ROWS, 16), lambda g: (g, 0)),
            pl.BlockSpec((TC_ROWS, 1), lambda g: (g, 0)),
            pl.BlockSpec((DIM_NODE_PAD, DIM_VSA), lambda g: (0, 0)),
            pl.BlockSpec((16, DIM_VSA), lambda g: (0, 0)),
            pl.BlockSpec((1, DIM_VSA), lambda g: (0, 0)),
        ],
        out_specs=pl.BlockSpec((TC_ROWS, DIM_VSA), lambda g: (g, 0)),
        out_shape=jax.ShapeDtypeStruct((rows, DIM_VSA), jnp.float32),
        scratch_shapes=[pltpu.VMEM((TC_ROWS, DIM_VSA), jnp.float32)],
        compiler_params=pltpu.CompilerParams(
            dimension_semantics=("arbitrary",),
        ),
    )(src_rows, dst_rows, edge_rows, valid2, pa_pad, pb_pad, fb)


def kernel(atoms, pairs, pair_mask, active, atom_projection, bond_projection,
           frequency_basis):
    atoms_pad = jnp.pad(atoms, ((0, 0), (0, 0), (0, DIM_NODE_PAD - DIM_NODE)))
    atoms_flat = jnp.pad(atoms_pad.reshape(B * N, DIM_NODE_PAD),
                         ((0, 8), (0, 0)))
    pairs_rows = jnp.transpose(pairs, (0, 3, 1, 2)).reshape(
        B * DIM_EDGE * N, N)
    pa_pad = jnp.pad(atom_projection, ((0, DIM_NODE_PAD - DIM_NODE), (0, 0)))
    pb_pad = jnp.pad(bond_projection, ((0, 16 - DIM_EDGE), (0, 0)))

    src_rows, dst_rows, edge_rows, valid = _sc_extract(pair_mask, atoms_flat,
                                                       pairs_rows)
    rows = B * MAX_EDGES
    out = _tc_compute(src_rows.reshape(rows, DIM_NODE_PAD),
                      dst_rows.reshape(rows, DIM_NODE_PAD),
                      edge_rows.reshape(rows, 16),
                      valid.reshape(rows, 1), pa_pad, pb_pad,
                      frequency_basis)
    return out.reshape(B, MAX_EDGES, DIM_VSA)
